# Initial kernel scaffold; baseline (speedup 1.0000x reference)
import jax
import jax.numpy as jnp
from jax.experimental import pallas as pl


def kernel(x):
    raise NotImplementedError("write your pallas kernel here")


# speedup vs baseline: 2.1866x; 2.1866x over previous
import functools

import jax
import jax.numpy as jnp
from jax import lax
from jax.experimental import pallas as pl

_N = 4096
_KS = (511, 1023, 1535, 2047, 2559, 3071, 3583)
_FRACS = (0.875, 0.75, 0.625, 0.5, 0.375, 0.25, 0.125)
_I32_MIN = -2147483648
_I32_MAX = 2147483647


def _key_to_f32(s):
    b = jnp.where(s < 0, s ^ jnp.int32(0x7FFFFFFF), s)
    return lax.bitcast_convert_type(b, jnp.float32)


def _select_pair(keys, k):
    rows = keys.shape[0]
    lo = jnp.full((rows, 1), _I32_MIN, jnp.int32)

    def body(i, lo):
        t = lo + (jnp.int32(1) << (jnp.int32(31) - i))
        cnt = jnp.sum((keys < t).astype(jnp.int32), axis=1, keepdims=True)
        return jnp.where(cnt >= k + 1, lo, t)

    s0 = lax.fori_loop(0, 32, body, lo)
    cnt_le = jnp.sum((keys <= s0).astype(jnp.int32), axis=1, keepdims=True)
    nxt = jnp.min(
        jnp.where(keys > s0, keys, jnp.int32(_I32_MAX)), axis=1, keepdims=True
    )
    s1 = jnp.where(cnt_le >= k + 2, s0, nxt)
    return s0, s1


def _body(x_ref, o_ref):
    x = x_ref[...]
    b = lax.bitcast_convert_type(x, jnp.int32)
    keys = jnp.where(b < 0, b ^ jnp.int32(0x7FFFFFFF), b)

    es = []
    for k, frac in zip(_KS, _FRACS):
        s0, s1 = _select_pair(keys, k)
        v0 = _key_to_f32(s0)
        v1 = _key_to_f32(s1)
        es.append(v0 * (1.0 - frac) + v1 * frac)
    e1, e2, e3, e4, e5, e6, e7 = es

    sa = e4
    sb = e6 - e2
    sg = (e6 + e2 - 2.0 * e4) / sb
    sk = (e7 - e5 + e3 - e1) / sb
    o_ref[...] = jnp.concatenate([sa, sb, sg, sk], axis=1)


@jax.jit
def kernel(x):
    n = x.shape[0]
    block_rows = 256
    out = pl.pallas_call(
        _body,
        grid=(n // block_rows,),
        in_specs=[pl.BlockSpec((block_rows, _N), lambda i: (i, 0))],
        out_specs=pl.BlockSpec((block_rows, 4), lambda i: (i, 0)),
        out_shape=jax.ShapeDtypeStruct((n, 4), x.dtype),
    )(x)
    return out
